# Initial kernel scaffold; baseline (speedup 1.0000x reference)
import jax
import jax.numpy as jnp
from jax.experimental import pallas as pl


def kernel(x, edge_index, W1, b1, g1, be1, W2, b2):
    raise NotImplementedError("write your pallas kernel here")


# speedup vs baseline: 24.0673x; 24.0673x over previous
import jax
import jax.numpy as jnp
from jax import lax
from jax.experimental import pallas as pl
from jax.experimental.pallas import tpu as pltpu, tpu_sc as plsc

BN_EPS = 1e-5

NC = 2
NS = 16
NW = NC * NS
C = 80

_SC_PARAMS = pltpu.CompilerParams(use_tc_tiling_on_sc=False)


def _pad_nodes(n):
    per = -(-n // NS)
    per = -(-per // 16) * 16
    return per * NS


def _identity_fill(idx_v, base, nck):
    lanes = lax.iota(jnp.int32, 16)

    def irow(i, carry):
        idx_v[i // 5, pl.ds((i % 5) * 16, 16)] = base + i * 16 + lanes
        return carry
    lax.fori_loop(0, nck * 5, irow, 0)


def _make_agg(N, E, D):
    EPW = E // NW
    CH = EPW // C
    assert CH * C == EPW and E % NW == 0
    NP_ = _pad_nodes(N)
    RPS = NP_ // NS
    NCK = RPS // C
    assert NCK * C == RPS
    mesh = plsc.VectorSubcoreMesh(core_axis_name="c", subcore_axis_name="s")

    def body(u_hbm, row_hbm, col_hbm, out_hbm, row_v, col_v, rows_v, zbuf,
             obuf, idx_v, acc_sh, sem):
        cid = lax.axis_index("c")
        sid = lax.axis_index("s")
        wid = sid * NC + cid

        def zrow(i, carry):
            for j in range(D // 16):
                zbuf[i, pl.ds(j * 16, 16)] = jnp.zeros((16,), jnp.float32)
            return carry
        lax.fori_loop(0, C, zrow, 0)
        _identity_fill(idx_v, sid * RPS, NCK)
        for k in range(NCK):
            pltpu.sync_copy(zbuf, acc_sh.at[idx_v.at[k]])
        plsc.subcore_barrier()

        pltpu.sync_copy(row_hbm.at[wid], row_v)
        pltpu.sync_copy(col_hbm.at[wid], col_v)

        def step(j, carry):
            pltpu.async_copy(u_hbm.at[row_v.at[j]], rows_v, sem).wait()
            pltpu.sync_copy(rows_v, acc_sh.at[col_v.at[j]], add=True)
            return carry
        lax.fori_loop(0, CH, step, 0)
        plsc.subcore_barrier()

        pltpu.sync_copy(acc_sh.at[pl.ds(sid * RPS, RPS)], obuf)
        pltpu.sync_copy(obuf, out_hbm.at[cid].at[sid])

    return pl.kernel(
        body,
        out_type=jax.ShapeDtypeStruct((NC, NS, RPS, D), jnp.float32),
        mesh=mesh,
        compiler_params=_SC_PARAMS,
        scratch_types=[
            pltpu.VMEM((CH, C), jnp.int32),
            pltpu.VMEM((CH, C), jnp.int32),
            pltpu.VMEM((C, D), jnp.float32),
            pltpu.VMEM((C, D), jnp.float32),
            pltpu.VMEM((RPS, D), jnp.float32),
            pltpu.VMEM((NCK, C), jnp.int32),
            pltpu.VMEM_SHARED((NP_, D), jnp.float32),
            pltpu.SemaphoreType.DMA,
        ],
    ), NP_


def _make_deg(N, E):
    DW = 16
    EPW = E // NW
    CH = EPW // C
    NP_ = _pad_nodes(N)
    RPS = NP_ // NS
    NCK = RPS // C
    mesh = plsc.VectorSubcoreMesh(core_axis_name="c", subcore_axis_name="s")

    def body(col_hbm, out_hbm, col_v, ones_v, zbuf, obuf, idx_v, acc_sh):
        cid = lax.axis_index("c")
        sid = lax.axis_index("s")
        wid = sid * NC + cid

        def fill(i, carry):
            ones_v[i, pl.ds(0, 16)] = jnp.ones((16,), jnp.float32)
            zbuf[i, pl.ds(0, 16)] = jnp.zeros((16,), jnp.float32)
            return carry
        lax.fori_loop(0, C, fill, 0)
        _identity_fill(idx_v, sid * RPS, NCK)
        for k in range(NCK):
            pltpu.sync_copy(zbuf, acc_sh.at[idx_v.at[k]])
        plsc.subcore_barrier()

        pltpu.sync_copy(col_hbm.at[wid], col_v)

        def step(j, carry):
            pltpu.sync_copy(ones_v, acc_sh.at[col_v.at[j]], add=True)
            return carry
        lax.fori_loop(0, CH, step, 0)
        plsc.subcore_barrier()

        pltpu.sync_copy(acc_sh.at[pl.ds(sid * RPS, RPS)], obuf)
        pltpu.sync_copy(obuf, out_hbm.at[cid].at[sid])

    return pl.kernel(
        body,
        out_type=jax.ShapeDtypeStruct((NC, NS, RPS, DW), jnp.float32),
        mesh=mesh,
        compiler_params=_SC_PARAMS,
        scratch_types=[
            pltpu.VMEM((CH, C), jnp.int32),
            pltpu.VMEM((C, DW), jnp.float32),
            pltpu.VMEM((C, DW), jnp.float32),
            pltpu.VMEM((RPS, DW), jnp.float32),
            pltpu.VMEM((NCK, C), jnp.int32),
            pltpu.VMEM_SHARED((NP_, DW), jnp.float32),
        ],
    ), NP_


BLK = 1000


def _tc1_body(x_ref, w_ref, g_ref, deg_ref, u_ref):
    s = g_ref[...] * jax.lax.rsqrt(jnp.float32(1.0 + BN_EPS))
    d = deg_ref[0, :, :1] + deg_ref[1, :, :1] + 1.0
    dinv = jax.lax.rsqrt(d)
    h = jnp.dot(x_ref[...], w_ref[...] * s,
                preferred_element_type=jnp.float32)
    u_ref[...] = h * dinv


def _tc2_body(agg_ref, u1_ref, deg_ref, g_ref, b1_ref, be1_ref, w2_ref,
              u2_ref):
    s = g_ref[...] * jax.lax.rsqrt(jnp.float32(1.0 + BN_EPS))
    b1p = b1_ref[...] * s + be1_ref[...]
    d = deg_ref[0, :, :1] + deg_ref[1, :, :1] + 1.0
    dinv = jax.lax.rsqrt(d)
    t = (agg_ref[0] + agg_ref[1] + u1_ref[...]) * dinv + b1p
    t = jnp.maximum(t, 0.0)
    h2 = jnp.dot(t, w2_ref[...], preferred_element_type=jnp.float32)
    u2_ref[...] = h2 * dinv


def _tc3_body(agg_ref, u2_ref, deg_ref, b2_ref, out_ref):
    d = deg_ref[0, :, :1] + deg_ref[1, :, :1] + 1.0
    dinv = jax.lax.rsqrt(d)
    out_ref[...] = (agg_ref[0] + agg_ref[1] + u2_ref[...]) * dinv \
        + b2_ref[...]


def _row_block(i):
    return (i, 0)


@jax.jit
def _run(x, edge_index, W1, b1, g1, be1, W2, b2):
    N, IN_DIM = x.shape
    HID = W1.shape[1]
    OUT = W2.shape[1]
    E = edge_index.shape[1]

    row3d = edge_index[0].reshape(NW, E // (NW * C), C)
    col3d = edge_index[1].reshape(NW, E // (NW * C), C)

    deg_k, NP_ = _make_deg(N, E)
    deg = deg_k(col3d).reshape(NC, NP_, 16)[:, :N]

    grid = N // BLK
    u1 = pl.pallas_call(
        _tc1_body,
        grid=(grid,),
        in_specs=[
            pl.BlockSpec((BLK, IN_DIM), _row_block),
            pl.BlockSpec((IN_DIM, HID), lambda i: (0, 0)),
            pl.BlockSpec((1, HID), lambda i: (0, 0)),
            pl.BlockSpec((2, BLK, 16), lambda i: (0, i, 0)),
        ],
        out_specs=pl.BlockSpec((BLK, HID), _row_block),
        out_shape=jax.ShapeDtypeStruct((N, HID), jnp.float32),
    )(x, W1, g1.reshape(1, HID), deg)

    agg1_k, _ = _make_agg(N, E, HID)
    agg1 = agg1_k(u1, row3d, col3d).reshape(NC, NP_, HID)[:, :N]

    u2 = pl.pallas_call(
        _tc2_body,
        grid=(grid,),
        in_specs=[
            pl.BlockSpec((2, BLK, HID), lambda i: (0, i, 0)),
            pl.BlockSpec((BLK, HID), _row_block),
            pl.BlockSpec((2, BLK, 16), lambda i: (0, i, 0)),
            pl.BlockSpec((1, HID), lambda i: (0, 0)),
            pl.BlockSpec((1, HID), lambda i: (0, 0)),
            pl.BlockSpec((1, HID), lambda i: (0, 0)),
            pl.BlockSpec((HID, OUT), lambda i: (0, 0)),
        ],
        out_specs=pl.BlockSpec((BLK, OUT), _row_block),
        out_shape=jax.ShapeDtypeStruct((N, OUT), jnp.float32),
    )(agg1, u1, deg, g1.reshape(1, HID), b1.reshape(1, HID),
      be1.reshape(1, HID), W2)

    agg2_k, _ = _make_agg(N, E, OUT)
    agg2 = agg2_k(u2, row3d, col3d).reshape(NC, NP_, OUT)[:, :N]

    out = pl.pallas_call(
        _tc3_body,
        grid=(grid,),
        in_specs=[
            pl.BlockSpec((2, BLK, OUT), lambda i: (0, i, 0)),
            pl.BlockSpec((BLK, OUT), _row_block),
            pl.BlockSpec((2, BLK, 16), lambda i: (0, i, 0)),
            pl.BlockSpec((1, OUT), lambda i: (0, 0)),
        ],
        out_specs=pl.BlockSpec((BLK, OUT), _row_block),
        out_shape=jax.ShapeDtypeStruct((N, OUT), jnp.float32),
    )(agg2, u2, deg, b2.reshape(1, OUT))

    return out


def kernel(x, edge_index, W1, b1, g1, be1, W2, b2):
    return _run(x, edge_index, W1, b1, g1, be1, W2, b2)


# speedup vs baseline: 39.0636x; 1.6231x over previous
import jax
import jax.numpy as jnp
from jax import lax
from jax.experimental import pallas as pl
from jax.experimental.pallas import tpu as pltpu, tpu_sc as plsc

BN_EPS = 1e-5

NC = 2
NS = 16
NW = NC * NS
C = 80

_SC_PARAMS = pltpu.CompilerParams(use_tc_tiling_on_sc=False)


def _pad_nodes(n):
    per = -(-n // NS)
    per = -(-per // 16) * 16
    return per * NS


def _identity_fill(idx_v, base, nck):
    lanes = lax.iota(jnp.int32, 16)

    def irow(i, carry):
        idx_v[i // 5, pl.ds((i % 5) * 16, 16)] = base + i * 16 + lanes
        return carry
    lax.fori_loop(0, nck * 5, irow, 0)


def _make_agg(N, E, D):
    EPW = E // NW
    CH = EPW // C
    assert CH * C == EPW and E % NW == 0
    NP_ = _pad_nodes(N)
    RPS = NP_ // NS
    NCK = RPS // C
    assert NCK * C == RPS
    mesh = plsc.VectorSubcoreMesh(core_axis_name="c", subcore_axis_name="s")

    W = 5
    NWIN = CH // W
    assert NWIN * W == CH and NWIN % 2 == 1

    def body(u_hbm, row_hbm, col_hbm, out_hbm, row_v, col_v, idx_v, *rest):
        bufs = rest[:2 * W]
        gsem, ssem, acc_sh_ref = rest[2 * W:]
        A, B = bufs[:W], bufs[W:]
        cid = lax.axis_index("c")
        sid = lax.axis_index("s")
        wid = sid * NC + cid

        def zrow(i, carry):
            for j in range(D // 16):
                A[0][i, pl.ds(j * 16, 16)] = jnp.zeros((16,), jnp.float32)
            return carry
        lax.fori_loop(0, C, zrow, 0)
        _identity_fill(idx_v, sid * RPS, NCK)
        for k in range(NCK):
            pltpu.sync_copy(A[0], acc_sh_ref.at[idx_v.at[k]])
        plsc.subcore_barrier()

        pltpu.sync_copy(row_hbm.at[wid], row_v)
        pltpu.sync_copy(col_hbm.at[wid], col_v)

        def fire_g(bset, w):
            for b in range(W):
                pltpu.async_copy(u_hbm.at[row_v.at[w * W + b]], bset[b], gsem)

        def drain_g(bset):
            for b in range(W):
                pltpu.make_async_copy(
                    u_hbm.at[row_v.at[0]], bset[b], gsem).wait()

        def fire_s(bset, w):
            for b in range(W):
                pltpu.async_copy(bset[b], acc_sh_ref.at[col_v.at[w * W + b]],
                                 ssem, add=True)

        def drain_s(bset):
            for b in range(W):
                pltpu.make_async_copy(
                    bset[b], acc_sh_ref.at[col_v.at[0]], ssem).wait()

        fire_g(A, 0)

        def step(t, carry):
            wA = 2 * t
            drain_g(A)
            fire_g(B, wA + 1)
            fire_s(A, wA)
            drain_s(A)
            drain_g(B)
            fire_g(A, wA + 2)
            fire_s(B, wA + 1)
            drain_s(B)
            return carry
        lax.fori_loop(0, (NWIN - 1) // 2, step, 0)
        drain_g(A)
        fire_s(A, NWIN - 1)
        drain_s(A)
        plsc.subcore_barrier()

        for k in range(NCK):
            pltpu.sync_copy(acc_sh_ref.at[pl.ds(sid * RPS + k * C, C)], A[0])
            pltpu.sync_copy(A[0], out_hbm.at[cid].at[sid].at[pl.ds(k * C, C)])

    return pl.kernel(
        body,
        out_type=jax.ShapeDtypeStruct((NC, NS, RPS, D), jnp.float32),
        mesh=mesh,
        compiler_params=_SC_PARAMS,
        scratch_types=[
            pltpu.VMEM((CH, C), jnp.int32),
            pltpu.VMEM((CH, C), jnp.int32),
            pltpu.VMEM((NCK, C), jnp.int32),
            *[pltpu.VMEM((C, D), jnp.float32) for _ in range(2 * W)],
            pltpu.SemaphoreType.DMA,
            pltpu.SemaphoreType.DMA,
            pltpu.VMEM_SHARED((NP_, D), jnp.float32),
        ],
    ), NP_


def _make_deg(N, E):
    DW = 16
    EPW = E // NW
    CH = EPW // C
    NP_ = _pad_nodes(N)
    RPS = NP_ // NS
    NCK = RPS // C
    mesh = plsc.VectorSubcoreMesh(core_axis_name="c", subcore_axis_name="s")

    def body(col_hbm, out_hbm, col_v, ones_v, zbuf, obuf, idx_v, acc_sh):
        cid = lax.axis_index("c")
        sid = lax.axis_index("s")
        wid = sid * NC + cid

        def fill(i, carry):
            ones_v[i, pl.ds(0, 16)] = jnp.ones((16,), jnp.float32)
            zbuf[i, pl.ds(0, 16)] = jnp.zeros((16,), jnp.float32)
            return carry
        lax.fori_loop(0, C, fill, 0)
        _identity_fill(idx_v, sid * RPS, NCK)
        for k in range(NCK):
            pltpu.sync_copy(zbuf, acc_sh.at[idx_v.at[k]])
        plsc.subcore_barrier()

        pltpu.sync_copy(col_hbm.at[wid], col_v)

        def step(j, carry):
            pltpu.sync_copy(ones_v, acc_sh.at[col_v.at[j]], add=True)
            return carry
        lax.fori_loop(0, CH, step, 0)
        plsc.subcore_barrier()

        pltpu.sync_copy(acc_sh.at[pl.ds(sid * RPS, RPS)], obuf)
        pltpu.sync_copy(obuf, out_hbm.at[cid].at[sid])

    return pl.kernel(
        body,
        out_type=jax.ShapeDtypeStruct((NC, NS, RPS, DW), jnp.float32),
        mesh=mesh,
        compiler_params=_SC_PARAMS,
        scratch_types=[
            pltpu.VMEM((CH, C), jnp.int32),
            pltpu.VMEM((C, DW), jnp.float32),
            pltpu.VMEM((C, DW), jnp.float32),
            pltpu.VMEM((RPS, DW), jnp.float32),
            pltpu.VMEM((NCK, C), jnp.int32),
            pltpu.VMEM_SHARED((NP_, DW), jnp.float32),
        ],
    ), NP_


BLK = 1000


def _tc1_body(x_ref, w_ref, g_ref, deg_ref, u_ref):
    s = g_ref[...] * jax.lax.rsqrt(jnp.float32(1.0 + BN_EPS))
    d = deg_ref[0, :, :1] + deg_ref[1, :, :1] + 1.0
    dinv = jax.lax.rsqrt(d)
    h = jnp.dot(x_ref[...], w_ref[...] * s,
                preferred_element_type=jnp.float32)
    u_ref[...] = h * dinv


def _tc2_body(agg_ref, u1_ref, deg_ref, g_ref, b1_ref, be1_ref, w2_ref,
              u2_ref):
    s = g_ref[...] * jax.lax.rsqrt(jnp.float32(1.0 + BN_EPS))
    b1p = b1_ref[...] * s + be1_ref[...]
    d = deg_ref[0, :, :1] + deg_ref[1, :, :1] + 1.0
    dinv = jax.lax.rsqrt(d)
    t = (agg_ref[0] + agg_ref[1] + u1_ref[...]) * dinv + b1p
    t = jnp.maximum(t, 0.0)
    h2 = jnp.dot(t, w2_ref[...], preferred_element_type=jnp.float32)
    u2_ref[...] = h2 * dinv


def _tc3_body(agg_ref, u2_ref, deg_ref, b2_ref, out_ref):
    d = deg_ref[0, :, :1] + deg_ref[1, :, :1] + 1.0
    dinv = jax.lax.rsqrt(d)
    out_ref[...] = (agg_ref[0] + agg_ref[1] + u2_ref[...]) * dinv \
        + b2_ref[...]


def _row_block(i):
    return (i, 0)


@jax.jit
def _run(x, edge_index, W1, b1, g1, be1, W2, b2):
    N, IN_DIM = x.shape
    HID = W1.shape[1]
    OUT = W2.shape[1]
    E = edge_index.shape[1]

    row3d = edge_index[0].reshape(NW, E // (NW * C), C)
    col3d = edge_index[1].reshape(NW, E // (NW * C), C)

    deg_k, NP_ = _make_deg(N, E)
    deg = deg_k(col3d).reshape(NC, NP_, 16)[:, :N]

    grid = N // BLK
    u1 = pl.pallas_call(
        _tc1_body,
        grid=(grid,),
        in_specs=[
            pl.BlockSpec((BLK, IN_DIM), _row_block),
            pl.BlockSpec((IN_DIM, HID), lambda i: (0, 0)),
            pl.BlockSpec((1, HID), lambda i: (0, 0)),
            pl.BlockSpec((2, BLK, 16), lambda i: (0, i, 0)),
        ],
        out_specs=pl.BlockSpec((BLK, HID), _row_block),
        out_shape=jax.ShapeDtypeStruct((N, HID), jnp.float32),
    )(x, W1, g1.reshape(1, HID), deg)

    agg1_k, _ = _make_agg(N, E, HID)
    agg1 = agg1_k(u1, row3d, col3d).reshape(NC, NP_, HID)[:, :N]

    u2 = pl.pallas_call(
        _tc2_body,
        grid=(grid,),
        in_specs=[
            pl.BlockSpec((2, BLK, HID), lambda i: (0, i, 0)),
            pl.BlockSpec((BLK, HID), _row_block),
            pl.BlockSpec((2, BLK, 16), lambda i: (0, i, 0)),
            pl.BlockSpec((1, HID), lambda i: (0, 0)),
            pl.BlockSpec((1, HID), lambda i: (0, 0)),
            pl.BlockSpec((1, HID), lambda i: (0, 0)),
            pl.BlockSpec((HID, OUT), lambda i: (0, 0)),
        ],
        out_specs=pl.BlockSpec((BLK, OUT), _row_block),
        out_shape=jax.ShapeDtypeStruct((N, OUT), jnp.float32),
    )(agg1, u1, deg, g1.reshape(1, HID), b1.reshape(1, HID),
      be1.reshape(1, HID), W2)

    agg2_k, _ = _make_agg(N, E, OUT)
    agg2 = agg2_k(u2, row3d, col3d).reshape(NC, NP_, OUT)[:, :N]

    out = pl.pallas_call(
        _tc3_body,
        grid=(grid,),
        in_specs=[
            pl.BlockSpec((2, BLK, OUT), lambda i: (0, i, 0)),
            pl.BlockSpec((BLK, OUT), _row_block),
            pl.BlockSpec((2, BLK, 16), lambda i: (0, i, 0)),
            pl.BlockSpec((1, OUT), lambda i: (0, 0)),
        ],
        out_specs=pl.BlockSpec((BLK, OUT), _row_block),
        out_shape=jax.ShapeDtypeStruct((N, OUT), jnp.float32),
    )(agg2, u2, deg, b2.reshape(1, OUT))

    return out


def kernel(x, edge_index, W1, b1, g1, be1, W2, b2):
    return _run(x, edge_index, W1, b1, g1, be1, W2, b2)


# speedup vs baseline: 46.2403x; 1.1837x over previous
import jax
import jax.numpy as jnp
from jax import lax
from jax.experimental import pallas as pl
from jax.experimental.pallas import tpu as pltpu, tpu_sc as plsc

BN_EPS = 1e-5

NC = 2
NS = 16
NW = NC * NS
C = 80

_SC_PARAMS = pltpu.CompilerParams(use_tc_tiling_on_sc=False)


def _pad_nodes(n):
    per = -(-n // NS)
    per = -(-per // 16) * 16
    return per * NS


def _identity_fill(idx_v, base, nck):
    lanes = lax.iota(jnp.int32, 16)

    def irow(i, carry):
        idx_v[i // 5, pl.ds((i % 5) * 16, 16)] = base + i * 16 + lanes
        return carry
    lax.fori_loop(0, nck * 5, irow, 0)


def _make_agg(N, E, D):
    EPW = E // NW
    CH = EPW // C
    assert CH * C == EPW and E % NW == 0
    NP_ = _pad_nodes(N)
    RPS = NP_ // NS
    NCK = RPS // C
    assert NCK * C == RPS
    mesh = plsc.VectorSubcoreMesh(core_axis_name="c", subcore_axis_name="s")

    W = 5
    NWIN = CH // W
    assert NWIN * W == CH and NWIN % 2 == 1

    def body(u_hbm, edge_hbm, out_hbm, row_v, col_v, idx_v, *rest):
        bufs = rest[:2 * W]
        gsem, ssem, acc_sh_ref = rest[2 * W:]
        A, B = bufs[:W], bufs[W:]
        cid = lax.axis_index("c")
        sid = lax.axis_index("s")
        wid = sid * NC + cid

        def zrow(i, carry):
            for j in range(D // 16):
                A[0][i, pl.ds(j * 16, 16)] = jnp.zeros((16,), jnp.float32)
            return carry
        lax.fori_loop(0, C, zrow, 0)
        _identity_fill(idx_v, sid * RPS, NCK)
        for k in range(NCK):
            pltpu.sync_copy(A[0], acc_sh_ref.at[idx_v.at[k]])
        plsc.subcore_barrier()

        pltpu.sync_copy(edge_hbm.at[0].at[wid], row_v)
        pltpu.sync_copy(edge_hbm.at[1].at[wid], col_v)

        def fire_g(bset, w):
            for b in range(W):
                pltpu.async_copy(u_hbm.at[row_v.at[w * W + b]], bset[b], gsem)

        def drain_g(bset):
            for b in range(W):
                pltpu.make_async_copy(
                    u_hbm.at[row_v.at[0]], bset[b], gsem).wait()

        def fire_s(bset, w):
            for b in range(W):
                pltpu.async_copy(bset[b], acc_sh_ref.at[col_v.at[w * W + b]],
                                 ssem, add=True)

        def drain_s(bset):
            for b in range(W):
                pltpu.make_async_copy(
                    bset[b], acc_sh_ref.at[col_v.at[0]], ssem).wait()

        fire_g(A, 0)

        def step(t, carry):
            wA = 2 * t
            drain_g(A)
            fire_g(B, wA + 1)
            fire_s(A, wA)
            drain_s(A)
            drain_g(B)
            fire_g(A, wA + 2)
            fire_s(B, wA + 1)
            drain_s(B)
            return carry
        lax.fori_loop(0, (NWIN - 1) // 2, step, 0)
        drain_g(A)
        fire_s(A, NWIN - 1)
        drain_s(A)
        plsc.subcore_barrier()

        for k in range(NCK):
            pltpu.sync_copy(acc_sh_ref.at[pl.ds(sid * RPS + k * C, C)], A[0])
            pltpu.sync_copy(A[0], out_hbm.at[cid].at[sid].at[pl.ds(k * C, C)])

    return pl.kernel(
        body,
        out_type=jax.ShapeDtypeStruct((NC, NS, RPS, D), jnp.float32),
        mesh=mesh,
        compiler_params=_SC_PARAMS,
        scratch_types=[
            pltpu.VMEM((CH, C), jnp.int32),
            pltpu.VMEM((CH, C), jnp.int32),
            pltpu.VMEM((NCK, C), jnp.int32),
            *[pltpu.VMEM((C, D), jnp.float32) for _ in range(2 * W)],
            pltpu.SemaphoreType.DMA,
            pltpu.SemaphoreType.DMA,
            pltpu.VMEM_SHARED((NP_, D), jnp.float32),
        ],
    ), NP_


def _make_deg(N, E):
    DW = 16
    EPW = E // NW
    CH = EPW // C
    NP_ = _pad_nodes(N)
    RPS = NP_ // NS
    NCK = RPS // C
    mesh = plsc.VectorSubcoreMesh(core_axis_name="c", subcore_axis_name="s")

    def body(edge_hbm, out_hbm, col_v, ones_v, zbuf, obuf, idx_v, acc_sh,
             ssem):
        cid = lax.axis_index("c")
        sid = lax.axis_index("s")
        wid = sid * NC + cid

        def fill(i, carry):
            ones_v[i, pl.ds(0, 16)] = jnp.ones((16,), jnp.float32)
            zbuf[i, pl.ds(0, 16)] = jnp.zeros((16,), jnp.float32)
            return carry
        lax.fori_loop(0, C, fill, 0)
        _identity_fill(idx_v, sid * RPS, NCK)
        for k in range(NCK):
            pltpu.sync_copy(zbuf, acc_sh.at[idx_v.at[k]])
        plsc.subcore_barrier()

        pltpu.sync_copy(edge_hbm.at[1].at[wid], col_v)

        def step(t, carry):
            for b in range(5):
                pltpu.async_copy(ones_v, acc_sh.at[col_v.at[t * 5 + b]],
                                 ssem, add=True)
            for b in range(5):
                pltpu.make_async_copy(ones_v, acc_sh.at[col_v.at[0]],
                                      ssem).wait()
            return carry
        lax.fori_loop(0, CH // 5, step, 0)
        plsc.subcore_barrier()

        pltpu.sync_copy(acc_sh.at[pl.ds(sid * RPS, RPS)], obuf)
        pltpu.sync_copy(obuf, out_hbm.at[cid].at[sid])

    return pl.kernel(
        body,
        out_type=jax.ShapeDtypeStruct((NC, NS, RPS, DW), jnp.float32),
        mesh=mesh,
        compiler_params=_SC_PARAMS,
        scratch_types=[
            pltpu.VMEM((CH, C), jnp.int32),
            pltpu.VMEM((C, DW), jnp.float32),
            pltpu.VMEM((C, DW), jnp.float32),
            pltpu.VMEM((RPS, DW), jnp.float32),
            pltpu.VMEM((NCK, C), jnp.int32),
            pltpu.VMEM_SHARED((NP_, DW), jnp.float32),
            pltpu.SemaphoreType.DMA,
        ],
    ), NP_


BLK = 1000


def _tc1_body(x_ref, w_ref, g_ref, deg_ref, u_ref):
    s = g_ref[...] * jax.lax.rsqrt(jnp.float32(1.0 + BN_EPS))
    d = deg_ref[0, :, :1] + deg_ref[1, :, :1] + 1.0
    dinv = jax.lax.rsqrt(d)
    h = jnp.dot(x_ref[...], w_ref[...] * s,
                preferred_element_type=jnp.float32)
    u_ref[...] = h * dinv


def _tc2_body(agg_ref, u1_ref, deg_ref, g_ref, b1_ref, be1_ref, w2_ref,
              u2_ref):
    s = g_ref[...] * jax.lax.rsqrt(jnp.float32(1.0 + BN_EPS))
    b1p = b1_ref[...] * s + be1_ref[...]
    d = deg_ref[0, :, :1] + deg_ref[1, :, :1] + 1.0
    dinv = jax.lax.rsqrt(d)
    t = (agg_ref[0] + agg_ref[1] + u1_ref[...]) * dinv + b1p
    t = jnp.maximum(t, 0.0)
    h2 = jnp.dot(t, w2_ref[...], preferred_element_type=jnp.float32)
    u2_ref[...] = h2 * dinv


def _tc3_body(agg_ref, u2_ref, deg_ref, b2_ref, out_ref):
    d = deg_ref[0, :, :1] + deg_ref[1, :, :1] + 1.0
    dinv = jax.lax.rsqrt(d)
    out_ref[...] = (agg_ref[0] + agg_ref[1] + u2_ref[...]) * dinv \
        + b2_ref[...]


def _row_block(i):
    return (i, 0)


@jax.jit
def _run(x, edge_index, W1, b1, g1, be1, W2, b2):
    N, IN_DIM = x.shape
    HID = W1.shape[1]
    OUT = W2.shape[1]
    E = edge_index.shape[1]

    edge3d = edge_index.reshape(2, NW, E // (NW * C), C)

    deg_k, NP_ = _make_deg(N, E)
    deg = deg_k(edge3d).reshape(NC, NP_, 16)

    grid = N // BLK
    u1 = pl.pallas_call(
        _tc1_body,
        grid=(grid,),
        in_specs=[
            pl.BlockSpec((BLK, IN_DIM), _row_block),
            pl.BlockSpec((IN_DIM, HID), lambda i: (0, 0)),
            pl.BlockSpec((1, HID), lambda i: (0, 0)),
            pl.BlockSpec((2, BLK, 16), lambda i: (0, i, 0)),
        ],
        out_specs=pl.BlockSpec((BLK, HID), _row_block),
        out_shape=jax.ShapeDtypeStruct((N, HID), jnp.float32),
    )(x, W1, g1.reshape(1, HID), deg)

    agg1_k, _ = _make_agg(N, E, HID)
    agg1 = agg1_k(u1, edge3d).reshape(NC, NP_, HID)

    u2 = pl.pallas_call(
        _tc2_body,
        grid=(grid,),
        in_specs=[
            pl.BlockSpec((2, BLK, HID), lambda i: (0, i, 0)),
            pl.BlockSpec((BLK, HID), _row_block),
            pl.BlockSpec((2, BLK, 16), lambda i: (0, i, 0)),
            pl.BlockSpec((1, HID), lambda i: (0, 0)),
            pl.BlockSpec((1, HID), lambda i: (0, 0)),
            pl.BlockSpec((1, HID), lambda i: (0, 0)),
            pl.BlockSpec((HID, OUT), lambda i: (0, 0)),
        ],
        out_specs=pl.BlockSpec((BLK, OUT), _row_block),
        out_shape=jax.ShapeDtypeStruct((N, OUT), jnp.float32),
    )(agg1, u1, deg, g1.reshape(1, HID), b1.reshape(1, HID),
      be1.reshape(1, HID), W2)

    agg2_k, _ = _make_agg(N, E, OUT)
    agg2 = agg2_k(u2, edge3d).reshape(NC, NP_, OUT)

    out = pl.pallas_call(
        _tc3_body,
        grid=(grid,),
        in_specs=[
            pl.BlockSpec((2, BLK, OUT), lambda i: (0, i, 0)),
            pl.BlockSpec((BLK, OUT), _row_block),
            pl.BlockSpec((2, BLK, 16), lambda i: (0, i, 0)),
            pl.BlockSpec((1, OUT), lambda i: (0, 0)),
        ],
        out_specs=pl.BlockSpec((BLK, OUT), _row_block),
        out_shape=jax.ShapeDtypeStruct((N, OUT), jnp.float32),
    )(agg2, u2, deg, b2.reshape(1, OUT))

    return out


def kernel(x, edge_index, W1, b1, g1, be1, W2, b2):
    return _run(x, edge_index, W1, b1, g1, be1, W2, b2)


# speedup vs baseline: 47.9724x; 1.0375x over previous
import jax
import jax.numpy as jnp
from jax import lax
from jax.experimental import pallas as pl
from jax.experimental.pallas import tpu as pltpu, tpu_sc as plsc

BN_EPS = 1e-5

NC = 2
NS = 16
NW = NC * NS
C = 80

_SC_PARAMS = pltpu.CompilerParams(use_tc_tiling_on_sc=False)


def _pad_nodes(n):
    per = -(-n // NS)
    per = -(-per // 16) * 16
    return per * NS


def _identity_fill(idx_v, base, nck):
    lanes = lax.iota(jnp.int32, 16)

    def irow(i, carry):
        idx_v[i // 5, pl.ds((i % 5) * 16, 16)] = base + i * 16 + lanes
        return carry
    lax.fori_loop(0, nck * 5, irow, 0)


def _make_agg(N, E, D):
    EPW = E // NW
    CH = EPW // C
    assert CH * C == EPW and E % NW == 0
    NP_ = _pad_nodes(N)
    RPS = NP_ // NS
    NCK = RPS // C
    assert NCK * C == RPS
    mesh = plsc.VectorSubcoreMesh(core_axis_name="c", subcore_axis_name="s")

    W = 5
    NWIN = CH // W
    assert NWIN * W == CH and NWIN % 2 == 1

    def body(u_hbm, edge_hbm, out_hbm, row_v, col_v, idx_v, *rest):
        bufs = rest[:2 * W]
        gsem, ssem, acc_sh_ref = rest[2 * W:]
        A, B = bufs[:W], bufs[W:]
        cid = lax.axis_index("c")
        sid = lax.axis_index("s")
        wid = sid * NC + cid

        pltpu.async_copy(edge_hbm.at[0].at[wid], row_v, gsem)
        pltpu.async_copy(edge_hbm.at[1].at[wid], col_v, gsem)

        def zrow(i, carry):
            for j in range(D // 16):
                A[0][i, pl.ds(j * 16, 16)] = jnp.zeros((16,), jnp.float32)
            return carry
        lax.fori_loop(0, C, zrow, 0)
        _identity_fill(idx_v, sid * RPS, NCK)
        for k in range(NCK):
            pltpu.async_copy(A[0], acc_sh_ref.at[idx_v.at[k]], ssem)
        for k in range(NCK):
            pltpu.make_async_copy(A[0], acc_sh_ref.at[idx_v.at[0]],
                                  ssem).wait()
        plsc.subcore_barrier()
        pltpu.make_async_copy(edge_hbm.at[0].at[wid], row_v, gsem).wait()
        pltpu.make_async_copy(edge_hbm.at[1].at[wid], col_v, gsem).wait()

        def fire_g(bset, w):
            for b in range(W):
                pltpu.async_copy(u_hbm.at[row_v.at[w * W + b]], bset[b], gsem)

        def drain_g(bset):
            for b in range(W):
                pltpu.make_async_copy(
                    u_hbm.at[row_v.at[0]], bset[b], gsem).wait()

        def fire_s(bset, w):
            for b in range(W):
                pltpu.async_copy(bset[b], acc_sh_ref.at[col_v.at[w * W + b]],
                                 ssem, add=True)

        def drain_s(bset):
            for b in range(W):
                pltpu.make_async_copy(
                    bset[b], acc_sh_ref.at[col_v.at[0]], ssem).wait()

        fire_g(A, 0)

        def step(t, carry):
            wA = 2 * t
            drain_g(A)
            fire_g(B, wA + 1)
            fire_s(A, wA)
            drain_s(A)
            drain_g(B)
            fire_g(A, wA + 2)
            fire_s(B, wA + 1)
            drain_s(B)
            return carry
        lax.fori_loop(0, (NWIN - 1) // 2, step, 0)
        drain_g(A)
        fire_s(A, NWIN - 1)
        drain_s(A)
        plsc.subcore_barrier()

        for k in range(NCK):
            pltpu.async_copy(acc_sh_ref.at[pl.ds(sid * RPS + k * C, C)],
                             bufs[k], gsem)
        for k in range(NCK):
            pltpu.make_async_copy(acc_sh_ref.at[pl.ds(sid * RPS, C)],
                                  bufs[k], gsem).wait()
        for k in range(NCK):
            pltpu.async_copy(bufs[k], out_hbm.at[cid].at[sid].at[
                pl.ds(k * C, C)], ssem)
        for k in range(NCK):
            pltpu.make_async_copy(bufs[k], out_hbm.at[cid].at[sid].at[
                pl.ds(0, C)], ssem).wait()

    return pl.kernel(
        body,
        out_type=jax.ShapeDtypeStruct((NC, NS, RPS, D), jnp.float32),
        mesh=mesh,
        compiler_params=_SC_PARAMS,
        scratch_types=[
            pltpu.VMEM((CH, C), jnp.int32),
            pltpu.VMEM((CH, C), jnp.int32),
            pltpu.VMEM((NCK, C), jnp.int32),
            *[pltpu.VMEM((C, D), jnp.float32) for _ in range(2 * W)],
            pltpu.SemaphoreType.DMA,
            pltpu.SemaphoreType.DMA,
            pltpu.VMEM_SHARED((NP_, D), jnp.float32),
        ],
    ), NP_


def _make_deg(N, E):
    DW = 16
    EPW = E // NW
    CH = EPW // C
    NP_ = _pad_nodes(N)
    RPS = NP_ // NS
    NCK = RPS // C
    mesh = plsc.VectorSubcoreMesh(core_axis_name="c", subcore_axis_name="s")

    def body(edge_hbm, out_hbm, col_v, ones_v, zbuf, obuf, idx_v, acc_sh,
             ssem):
        cid = lax.axis_index("c")
        sid = lax.axis_index("s")
        wid = sid * NC + cid

        def fill(i, carry):
            ones_v[i, pl.ds(0, 16)] = jnp.ones((16,), jnp.float32)
            zbuf[i, pl.ds(0, 16)] = jnp.zeros((16,), jnp.float32)
            return carry
        pltpu.async_copy(edge_hbm.at[1].at[wid], col_v, ssem)
        lax.fori_loop(0, C, fill, 0)
        _identity_fill(idx_v, sid * RPS, NCK)
        for k in range(NCK):
            pltpu.async_copy(zbuf, acc_sh.at[idx_v.at[k]], ssem)
        for k in range(NCK):
            pltpu.make_async_copy(zbuf, acc_sh.at[idx_v.at[0]], ssem).wait()
        pltpu.make_async_copy(edge_hbm.at[1].at[wid], col_v, ssem).wait()
        plsc.subcore_barrier()

        def step(t, carry):
            for b in range(5):
                pltpu.async_copy(ones_v, acc_sh.at[col_v.at[t * 5 + b]],
                                 ssem, add=True)
            for b in range(5):
                pltpu.make_async_copy(ones_v, acc_sh.at[col_v.at[0]],
                                      ssem).wait()
            return carry
        lax.fori_loop(0, CH // 5, step, 0)
        plsc.subcore_barrier()

        pltpu.sync_copy(acc_sh.at[pl.ds(sid * RPS, RPS)], obuf)
        pltpu.sync_copy(obuf, out_hbm.at[cid].at[sid])

    return pl.kernel(
        body,
        out_type=jax.ShapeDtypeStruct((NC, NS, RPS, DW), jnp.float32),
        mesh=mesh,
        compiler_params=_SC_PARAMS,
        scratch_types=[
            pltpu.VMEM((CH, C), jnp.int32),
            pltpu.VMEM((C, DW), jnp.float32),
            pltpu.VMEM((C, DW), jnp.float32),
            pltpu.VMEM((RPS, DW), jnp.float32),
            pltpu.VMEM((NCK, C), jnp.int32),
            pltpu.VMEM_SHARED((NP_, DW), jnp.float32),
            pltpu.SemaphoreType.DMA,
        ],
    ), NP_


BLK = 1000


def _tc1_body(x_ref, w_ref, g_ref, deg_ref, u_ref):
    s = g_ref[...] * jax.lax.rsqrt(jnp.float32(1.0 + BN_EPS))
    d = deg_ref[0, :, :1] + deg_ref[1, :, :1] + 1.0
    dinv = jax.lax.rsqrt(d)
    h = jnp.dot(x_ref[...], w_ref[...] * s,
                preferred_element_type=jnp.float32)
    u_ref[...] = h * dinv


def _tc2_body(agg_ref, u1_ref, deg_ref, g_ref, b1_ref, be1_ref, w2_ref,
              u2_ref):
    s = g_ref[...] * jax.lax.rsqrt(jnp.float32(1.0 + BN_EPS))
    b1p = b1_ref[...] * s + be1_ref[...]
    d = deg_ref[0, :, :1] + deg_ref[1, :, :1] + 1.0
    dinv = jax.lax.rsqrt(d)
    t = (agg_ref[0] + agg_ref[1] + u1_ref[...]) * dinv + b1p
    t = jnp.maximum(t, 0.0)
    h2 = jnp.dot(t, w2_ref[...], preferred_element_type=jnp.float32)
    u2_ref[...] = h2 * dinv


def _tc3_body(agg_ref, u2_ref, deg_ref, b2_ref, out_ref):
    d = deg_ref[0, :, :1] + deg_ref[1, :, :1] + 1.0
    dinv = jax.lax.rsqrt(d)
    out_ref[...] = (agg_ref[0] + agg_ref[1] + u2_ref[...]) * dinv \
        + b2_ref[...]


def _row_block(i):
    return (i, 0)


@jax.jit
def _run(x, edge_index, W1, b1, g1, be1, W2, b2):
    N, IN_DIM = x.shape
    HID = W1.shape[1]
    OUT = W2.shape[1]
    E = edge_index.shape[1]

    edge3d = edge_index.reshape(2, NW, E // (NW * C), C)

    deg_k, NP_ = _make_deg(N, E)
    deg = deg_k(edge3d).reshape(NC, NP_, 16)

    grid = N // BLK
    u1 = pl.pallas_call(
        _tc1_body,
        grid=(grid,),
        in_specs=[
            pl.BlockSpec((BLK, IN_DIM), _row_block),
            pl.BlockSpec((IN_DIM, HID), lambda i: (0, 0)),
            pl.BlockSpec((1, HID), lambda i: (0, 0)),
            pl.BlockSpec((2, BLK, 16), lambda i: (0, i, 0)),
        ],
        out_specs=pl.BlockSpec((BLK, HID), _row_block),
        out_shape=jax.ShapeDtypeStruct((N, HID), jnp.float32),
    )(x, W1, g1.reshape(1, HID), deg)

    agg1_k, _ = _make_agg(N, E, HID)
    agg1 = agg1_k(u1, edge3d).reshape(NC, NP_, HID)

    u2 = pl.pallas_call(
        _tc2_body,
        grid=(grid,),
        in_specs=[
            pl.BlockSpec((2, BLK, HID), lambda i: (0, i, 0)),
            pl.BlockSpec((BLK, HID), _row_block),
            pl.BlockSpec((2, BLK, 16), lambda i: (0, i, 0)),
            pl.BlockSpec((1, HID), lambda i: (0, 0)),
            pl.BlockSpec((1, HID), lambda i: (0, 0)),
            pl.BlockSpec((1, HID), lambda i: (0, 0)),
            pl.BlockSpec((HID, OUT), lambda i: (0, 0)),
        ],
        out_specs=pl.BlockSpec((BLK, OUT), _row_block),
        out_shape=jax.ShapeDtypeStruct((N, OUT), jnp.float32),
    )(agg1, u1, deg, g1.reshape(1, HID), b1.reshape(1, HID),
      be1.reshape(1, HID), W2)

    agg2_k, _ = _make_agg(N, E, OUT)
    agg2 = agg2_k(u2, edge3d).reshape(NC, NP_, OUT)

    out = pl.pallas_call(
        _tc3_body,
        grid=(grid,),
        in_specs=[
            pl.BlockSpec((2, BLK, OUT), lambda i: (0, i, 0)),
            pl.BlockSpec((BLK, OUT), _row_block),
            pl.BlockSpec((2, BLK, 16), lambda i: (0, i, 0)),
            pl.BlockSpec((1, OUT), lambda i: (0, 0)),
        ],
        out_specs=pl.BlockSpec((BLK, OUT), _row_block),
        out_shape=jax.ShapeDtypeStruct((N, OUT), jnp.float32),
    )(agg2, u2, deg, b2.reshape(1, OUT))

    return out


def kernel(x, edge_index, W1, b1, g1, be1, W2, b2):
    return _run(x, edge_index, W1, b1, g1, be1, W2, b2)
